# baseline (speedup 1.0000x reference)
import functools

import jax
import jax.numpy as jnp
from jax.experimental import pallas as pl
from jax.experimental.pallas import tpu as pltpu

BEV_H, BEV_W = 200, 200
RES = 0.5
X0, Y0 = -50.0, -50.0
EPS = 1e-5

N_PTS = 20000
CHUNK = 2000
S = 8
N_CHUNKS = N_PTS // CHUNK
ROWS = 1000
LANES = 256

_CP = pltpu.CompilerParams(
    dimension_semantics=("parallel",),
    vmem_limit_bytes=100 * 1024 * 1024,
)


def _scatter_kernel(rh_ref, ri_ref, xi_ref, vh_ref, vi_ref, out_ref, acc):
    step = pl.program_id(0)

    @pl.when(step == 0)
    def _init():
        acc[...] = jnp.zeros_like(acc)

    iota = jax.lax.broadcasted_iota(jnp.int32, (1, LANES), 1)

    def body(i, _):
        for k in range(S):
            idx = i * S + k
            r = rh_ref[0, 0, idx]
            ri = ri_ref[0, 0, idx]
            c = xi_ref[0, 0, idx]
            vh = vh_ref[0, 0, idx]
            vi = vi_ref[0, 0, idx]
            onehot = iota == c
            row = acc[k, pl.ds(r, 1), :]
            acc[k, pl.ds(r, 1), :] = jnp.where(onehot, jnp.maximum(row, vh), row)
            row2 = acc[k, pl.ds(ri, 1), :]
            acc[k, pl.ds(ri, 1), :] = jnp.where(onehot, jnp.maximum(row2, vi), row2)
        return ()

    jax.lax.fori_loop(0, CHUNK // S, body, ())

    @pl.when(step == N_CHUNKS - 1)
    def _reduce():
        m01 = jnp.maximum(acc[0], acc[1])
        m23 = jnp.maximum(acc[2], acc[3])
        m45 = jnp.maximum(acc[4], acc[5])
        m67 = jnp.maximum(acc[6], acc[7])
        out_ref[...] = jnp.maximum(jnp.maximum(m01, m23), jnp.maximum(m45, m67))


def _points_to_bev_pallas(points):
    x, y, z, inten = points[:, 0], points[:, 1], points[:, 2], points[:, 3]
    xi = jnp.clip(jnp.floor((x - X0) / RES).astype(jnp.int32), 0, BEV_W - 1)
    yi = jnp.clip(jnp.floor((y - Y0) / RES).astype(jnp.int32), 0, BEV_H - 1)
    hi = ((z > -2.0).astype(jnp.int32) + (z > 0.0).astype(jnp.int32)
          + (z > 2.0).astype(jnp.int32) + (z > 4.0).astype(jnp.int32))
    hi = jnp.clip(hi, 0, 3)
    rh = (hi * BEV_H + yi).reshape(N_CHUNKS, 1, CHUNK)
    ri = (800 + yi).reshape(N_CHUNKS, 1, CHUNK)
    xi3 = xi.reshape(N_CHUNKS, 1, CHUNK)
    vh = (z + 2.0).reshape(N_CHUNKS, 1, CHUNK)
    vi = inten.reshape(N_CHUNKS, 1, CHUNK)

    smem = pl.BlockSpec((1, 1, CHUNK), lambda i: (i, 0, 0),
                        memory_space=pltpu.SMEM)
    grid_out = pl.pallas_call(
        _scatter_kernel,
        grid=(N_CHUNKS,),
        in_specs=[smem, smem, smem, smem, smem],
        out_specs=pl.BlockSpec((ROWS, LANES), lambda i: (0, 0)),
        out_shape=jax.ShapeDtypeStruct((ROWS, LANES), jnp.float32),
        scratch_shapes=[pltpu.VMEM((S, ROWS, LANES), jnp.float32)],
    )(rh, ri, xi3, vh, vi)
    return grid_out[:, :BEV_W]


def _mm_kernel(x_ref, w_ref, b_ref, o_ref, *, relu):
    acc = jnp.dot(x_ref[...], w_ref[...], preferred_element_type=jnp.float32)
    acc = acc + b_ref[...]
    if relu:
        acc = jnp.maximum(acc, 0.0)
    o_ref[...] = acc.astype(o_ref.dtype)


def _mm(x, w, b, *, tm, relu, out_dtype=jnp.bfloat16):
    M, K = x.shape
    N = w.shape[1]
    return pl.pallas_call(
        functools.partial(_mm_kernel, relu=relu),
        grid=(M // tm,),
        in_specs=[pl.BlockSpec((tm, K), lambda i: (i, 0)),
                  pl.BlockSpec((K, N), lambda i: (0, 0)),
                  pl.BlockSpec((1, N), lambda i: (0, 0))],
        out_specs=pl.BlockSpec((tm, N), lambda i: (i, 0)),
        out_shape=jax.ShapeDtypeStruct((M, N), out_dtype),
        compiler_params=_CP,
    )(x.astype(jnp.bfloat16), w.astype(jnp.bfloat16),
      b.reshape(1, N).astype(jnp.float32))


def _mmn_kernel(x_ref, w_ref, o_ref):
    o_ref[...] = jnp.dot(x_ref[...], w_ref[...],
                         preferred_element_type=jnp.float32).astype(o_ref.dtype)


def _mm_ngrid(x, w, *, tn, out_dtype=jnp.bfloat16):
    M, K = x.shape
    N = w.shape[1]
    return pl.pallas_call(
        _mmn_kernel,
        grid=(N // tn,),
        in_specs=[pl.BlockSpec((M, K), lambda i: (0, 0)),
                  pl.BlockSpec((K, tn), lambda i: (0, i))],
        out_specs=pl.BlockSpec((M, tn), lambda i: (0, i)),
        out_shape=jax.ShapeDtypeStruct((M, N), out_dtype),
        compiler_params=_CP,
    )(x.astype(jnp.bfloat16), w.astype(jnp.bfloat16))


def _p1_kernel(x_ref, w_ref, b_ref, o_ref, patch):
    i = pl.program_id(0)
    for dy in range(3):
        for dx in range(3):
            t = dy * 3 + dx
            sl = x_ref[pl.ds(i * 16 + dy, 16), pl.ds(dx, 256), :]
            patch[:, t * 128:(t + 1) * 128] = sl.reshape(4096, 128)
    acc = jnp.dot(patch[...], w_ref[...], preferred_element_type=jnp.float32)
    acc = jnp.maximum(acc + b_ref[...], 0.0)
    o_ref[...] = acc.astype(o_ref.dtype)


def _p1_conv(xp, w, b):
    return pl.pallas_call(
        _p1_kernel,
        grid=(8,),
        in_specs=[pl.BlockSpec((130, 258, 128), lambda i: (0, 0, 0)),
                  pl.BlockSpec((1152, 256), lambda i: (0, 0)),
                  pl.BlockSpec((1, 256), lambda i: (0, 0))],
        out_specs=pl.BlockSpec((4096, 256), lambda i: (i, 0)),
        out_shape=jax.ShapeDtypeStruct((32768, 256), jnp.bfloat16),
        scratch_shapes=[pltpu.VMEM((4096, 1152), jnp.bfloat16)],
        compiler_params=_CP,
    )(xp, w.astype(jnp.bfloat16), b.reshape(1, 256).astype(jnp.float32))


def _fold_bn(conv_p, bn_p):
    s = bn_p["g"] * jax.lax.rsqrt(bn_p["v"] + EPS)
    w = conv_p["w"] * s[:, None, None, None]
    b = conv_p["b"] * s + bn_p["beta"] - bn_p["m"] * s
    return w, b


def _w_mat(w):
    return jnp.transpose(w, (2, 3, 1, 0)).reshape(-1, w.shape[0])


def _x9(x, stride):
    H, W, C = x.shape
    Ho, Wo = H // stride, W // stride
    xp = jnp.pad(x, ((1, 1), (1, 1), (0, 0)))
    taps = []
    for dy in range(3):
        for dx in range(3):
            taps.append(jax.lax.slice(
                xp, (dy, dx, 0),
                (dy + stride * (Ho - 1) + 1, dx + stride * (Wo - 1) + 1, C),
                (stride, stride, 1)))
    return jnp.concatenate(taps, axis=-1)


def _resize_mat(n_out, n_in):
    o = jnp.arange(n_out, dtype=jnp.float32)
    src = (o + 0.5) * (n_in / n_out) - 0.5
    i0 = jnp.floor(src)
    f = src - i0
    i0c = jnp.clip(i0.astype(jnp.int32), 0, n_in - 1)
    i1c = jnp.clip(i0.astype(jnp.int32) + 1, 0, n_in - 1)
    i = jnp.arange(n_in, dtype=jnp.int32)
    a = ((1.0 - f)[:, None] * (i[None, :] == i0c[:, None])
         + f[:, None] * (i[None, :] == i1c[:, None]))
    return a.astype(jnp.float32)


def _cam_branch(images, p):
    img = images[0].transpose(1, 2, 0).astype(jnp.bfloat16)

    w1, b1 = _fold_bn(p["c1"], p["bn1"])
    x = _x9(img, 2).reshape(512 * 1024, 27)
    y = _mm(x, _w_mat(w1), b1, tm=16384, relu=True)

    w2, b2 = _fold_bn(p["c2"], p["bn2"])
    x = _x9(y.reshape(512, 1024, 32), 2).reshape(256 * 512, 288)
    y = _mm(x, _w_mat(w2), b2, tm=8192, relu=True)

    w3, b3 = _fold_bn(p["c3"], p["bn3"])
    x = _x9(y.reshape(256, 512, 64), 2).reshape(128 * 256, 576)
    y = _mm(x, _w_mat(w3), b3, tm=4096, relu=True)

    wp1, bp1 = _fold_bn(p["p1"], p["pbn"])
    xp = jnp.pad(y.reshape(128, 256, 128), ((1, 1), (1, 1), (0, 0)))
    y = _p1_conv(xp, _w_mat(wp1), bp1)

    y = _mm(y, _w_mat(p["p2"]["w"]), p["p2"]["b"], tm=4096, relu=False)

    ah = _resize_mat(BEV_H, 128)
    aw = _resize_mat(BEV_W, 256)
    t = _mm_ngrid(ah, y.reshape(128, 256 * 128), tn=4096)
    t = t.reshape(200, 256, 128).transpose(1, 0, 2).reshape(256, 200 * 128)
    t = _mm_ngrid(aw, t, tn=3200)
    return t.reshape(200, 200, 128)


def _lid_branch(points, p):
    bev = _points_to_bev_pallas(points)
    bev = bev.reshape(5, 200, 200).transpose(1, 2, 0)

    w1, b1 = _fold_bn(p["c1"], p["bn1"])
    x = _x9(bev.astype(jnp.bfloat16), 1).reshape(40000, 45)
    y = _mm(x, _w_mat(w1), b1, tm=8000, relu=True)

    w2, b2 = _fold_bn(p["c2"], p["bn2"])
    x = _x9(y.reshape(200, 200, 32), 1).reshape(40000, 288)
    y = _mm(x, _w_mat(w2), b2, tm=8000, relu=True)

    y = _mm(y, _w_mat(p["c3"]["w"]), p["c3"]["b"], tm=8000, relu=False)
    return y.reshape(200, 200, 128)


def kernel(images, points, cam_params, lidar_params):
    cam = _cam_branch(images, cam_params)
    lid = _lid_branch(points, lidar_params)
    cam_chw = cam.transpose(2, 1, 0).astype(jnp.float32)
    lid_chw = lid.transpose(2, 0, 1).astype(jnp.float32)
    return jnp.concatenate([cam_chw, lid_chw], axis=0)[None]


# speedup vs baseline: 1.1506x; 1.1506x over previous
import functools

import jax
import jax.numpy as jnp
from jax.experimental import pallas as pl
from jax.experimental.pallas import tpu as pltpu

BEV_H, BEV_W = 200, 200
RES = 0.5
X0, Y0 = -50.0, -50.0
EPS = 1e-5

N_PTS = 20000
CHUNK = 2000
S = 8
N_CHUNKS = N_PTS // CHUNK
ROWS = 1000
LANES = 256

_CP = pltpu.CompilerParams(
    dimension_semantics=("parallel",),
    vmem_limit_bytes=100 * 1024 * 1024,
)


def _scatter_kernel(rh_ref, ri_ref, xi_ref, vh_ref, vi_ref, out_ref, acc):
    step = pl.program_id(0)

    @pl.when(step == 0)
    def _init():
        acc[...] = jnp.zeros_like(acc)

    iota = jax.lax.broadcasted_iota(jnp.int32, (1, LANES), 1)

    def body(i, _):
        for k in range(S):
            idx = i * S + k
            r = rh_ref[0, 0, idx]
            ri = ri_ref[0, 0, idx]
            c = xi_ref[0, 0, idx]
            vh = vh_ref[0, 0, idx]
            vi = vi_ref[0, 0, idx]
            onehot = iota == c
            row = acc[k, pl.ds(r, 1), :]
            acc[k, pl.ds(r, 1), :] = jnp.where(onehot, jnp.maximum(row, vh), row)
            row2 = acc[k, pl.ds(ri, 1), :]
            acc[k, pl.ds(ri, 1), :] = jnp.where(onehot, jnp.maximum(row2, vi), row2)
        return ()

    jax.lax.fori_loop(0, CHUNK // S, body, ())

    @pl.when(step == N_CHUNKS - 1)
    def _reduce():
        m01 = jnp.maximum(acc[0], acc[1])
        m23 = jnp.maximum(acc[2], acc[3])
        m45 = jnp.maximum(acc[4], acc[5])
        m67 = jnp.maximum(acc[6], acc[7])
        out_ref[...] = jnp.maximum(jnp.maximum(m01, m23), jnp.maximum(m45, m67))


def _points_to_bev_pallas(points):
    x, y, z, inten = points[:, 0], points[:, 1], points[:, 2], points[:, 3]
    xi = jnp.clip(jnp.floor((x - X0) / RES).astype(jnp.int32), 0, BEV_W - 1)
    yi = jnp.clip(jnp.floor((y - Y0) / RES).astype(jnp.int32), 0, BEV_H - 1)
    hi = ((z > -2.0).astype(jnp.int32) + (z > 0.0).astype(jnp.int32)
          + (z > 2.0).astype(jnp.int32) + (z > 4.0).astype(jnp.int32))
    hi = jnp.clip(hi, 0, 3)
    rh = (hi * BEV_H + yi).reshape(N_CHUNKS, 1, CHUNK)
    ri = (800 + yi).reshape(N_CHUNKS, 1, CHUNK)
    xi3 = xi.reshape(N_CHUNKS, 1, CHUNK)
    vh = (z + 2.0).reshape(N_CHUNKS, 1, CHUNK)
    vi = inten.reshape(N_CHUNKS, 1, CHUNK)

    smem = pl.BlockSpec((1, 1, CHUNK), lambda i: (i, 0, 0),
                        memory_space=pltpu.SMEM)
    grid_out = pl.pallas_call(
        _scatter_kernel,
        grid=(N_CHUNKS,),
        in_specs=[smem, smem, smem, smem, smem],
        out_specs=pl.BlockSpec((ROWS, LANES), lambda i: (0, 0)),
        out_shape=jax.ShapeDtypeStruct((ROWS, LANES), jnp.float32),
        scratch_shapes=[pltpu.VMEM((S, ROWS, LANES), jnp.float32)],
    )(rh, ri, xi3, vh, vi)
    return grid_out[:, :BEV_W]


def _mmt_kernel(w_ref, x_ref, b_ref, o_ref, *, relu):
    acc = jnp.dot(w_ref[...], x_ref[...], preferred_element_type=jnp.float32)
    acc = acc + b_ref[...][:, :1]
    if relu:
        acc = jnp.maximum(acc, 0.0)
    o_ref[...] = acc.astype(o_ref.dtype)


def _mmt(w, x, b, *, tn, relu, out_dtype=jnp.bfloat16):
    M, K = w.shape
    N = x.shape[1]
    bb = jnp.broadcast_to(b.astype(jnp.float32)[:, None], (M, 128))
    return pl.pallas_call(
        functools.partial(_mmt_kernel, relu=relu),
        grid=(N // tn,),
        in_specs=[pl.BlockSpec((M, K), lambda i: (0, 0)),
                  pl.BlockSpec((K, tn), lambda i: (0, i)),
                  pl.BlockSpec((M, 128), lambda i: (0, 0))],
        out_specs=pl.BlockSpec((M, tn), lambda i: (0, i)),
        out_shape=jax.ShapeDtypeStruct((M, N), out_dtype),
        compiler_params=_CP,
    )(w.astype(jnp.bfloat16), x.astype(jnp.bfloat16), bb)


def _resize_kernel(x_ref, ah_ref, awt_ref, o_ref):
    for c in range(16):
        xc = x_ref[pl.ds(c, 1), :, :].reshape(128, 256)
        t1 = jnp.dot(ah_ref[...], xc, preferred_element_type=jnp.float32)
        t2 = jnp.dot(t1.astype(jnp.bfloat16), awt_ref[...],
                     preferred_element_type=jnp.float32)
        o_ref[pl.ds(c, 1), :, :] = t2[None].astype(o_ref.dtype)


def _resize(x, ah, awt):
    return pl.pallas_call(
        _resize_kernel,
        grid=(8,),
        in_specs=[pl.BlockSpec((16, 128, 256), lambda i: (i, 0, 0)),
                  pl.BlockSpec((200, 128), lambda i: (0, 0)),
                  pl.BlockSpec((256, 200), lambda i: (0, 0))],
        out_specs=pl.BlockSpec((16, 200, 200), lambda i: (i, 0, 0)),
        out_shape=jax.ShapeDtypeStruct((128, 200, 200), jnp.bfloat16),
        compiler_params=_CP,
    )(x.astype(jnp.bfloat16), ah.astype(jnp.bfloat16), awt.astype(jnp.bfloat16))


def _fold_bn(conv_p, bn_p):
    s = bn_p["g"] * jax.lax.rsqrt(bn_p["v"] + EPS)
    w = conv_p["w"] * s[:, None, None, None]
    b = conv_p["b"] * s + bn_p["beta"] - bn_p["m"] * s
    return w, b


def _w_rows(w):
    return jnp.transpose(w, (0, 2, 3, 1)).reshape(w.shape[0], -1)


def _x9p(x, stride):
    C, H, W = x.shape
    Ho, Wo = H // stride, W // stride
    xp = jnp.pad(x, ((0, 0), (1, 1), (1, 1)))
    taps = []
    for dy in range(3):
        for dx in range(3):
            taps.append(jax.lax.slice(
                xp, (0, dy, dx),
                (C, dy + stride * (Ho - 1) + 1, dx + stride * (Wo - 1) + 1),
                (1, stride, stride)))
    return jnp.stack(taps, 0).reshape(9 * C, Ho * Wo)


def _resize_mat(n_out, n_in):
    o = jnp.arange(n_out, dtype=jnp.float32)
    src = (o + 0.5) * (n_in / n_out) - 0.5
    i0 = jnp.floor(src)
    f = src - i0
    i0c = jnp.clip(i0.astype(jnp.int32), 0, n_in - 1)
    i1c = jnp.clip(i0.astype(jnp.int32) + 1, 0, n_in - 1)
    i = jnp.arange(n_in, dtype=jnp.int32)
    a = ((1.0 - f)[:, None] * (i[None, :] == i0c[:, None])
         + f[:, None] * (i[None, :] == i1c[:, None]))
    return a.astype(jnp.float32)


def _cam_branch(images, p):
    img = images[0].astype(jnp.bfloat16)

    w1, b1 = _fold_bn(p["c1"], p["bn1"])
    y = _mmt(_w_rows(w1), _x9p(img, 2), b1, tn=16384, relu=True)

    w2, b2 = _fold_bn(p["c2"], p["bn2"])
    y = _mmt(_w_rows(w2), _x9p(y.reshape(32, 512, 1024), 2), b2,
             tn=8192, relu=True)

    w3, b3 = _fold_bn(p["c3"], p["bn3"])
    y = _mmt(_w_rows(w3), _x9p(y.reshape(64, 256, 512), 2), b3,
             tn=4096, relu=True)

    wp1, bp1 = _fold_bn(p["p1"], p["pbn"])
    y = _mmt(_w_rows(wp1), _x9p(y.reshape(128, 128, 256), 1), bp1,
             tn=4096, relu=True)

    y = _mmt(_w_rows(p["p2"]["w"]), y, p["p2"]["b"], tn=4096, relu=False)

    ah = _resize_mat(BEV_H, 128)
    awt = _resize_mat(BEV_W, 256).T
    return _resize(y.reshape(128, 128, 256), ah, awt)


def _lid_branch(points, p):
    bev = _points_to_bev_pallas(points).reshape(5, 200, 200)

    w1, b1 = _fold_bn(p["c1"], p["bn1"])
    y = _mmt(_w_rows(w1), _x9p(bev.astype(jnp.bfloat16), 1), b1,
             tn=40000, relu=True)

    w2, b2 = _fold_bn(p["c2"], p["bn2"])
    y = _mmt(_w_rows(w2), _x9p(y.reshape(32, 200, 200), 1), b2,
             tn=40000, relu=True)

    y = _mmt(_w_rows(p["c3"]["w"]), y, p["c3"]["b"], tn=40000, relu=False)
    return y.reshape(128, 200, 200)


def kernel(images, points, cam_params, lidar_params):
    cam = _cam_branch(images, cam_params)
    lid = _lid_branch(points, lidar_params)
    return jnp.concatenate([cam, lid], axis=0)[None].astype(jnp.float32)


# speedup vs baseline: 4.8106x; 4.1810x over previous
import functools

import jax
import jax.numpy as jnp
from jax.experimental import pallas as pl
from jax.experimental.pallas import tpu as pltpu

BEV_H, BEV_W = 200, 200
RES = 0.5
X0, Y0 = -50.0, -50.0
EPS = 1e-5

N_PTS = 20000
CHUNK = 2000
S = 8
N_CHUNKS = N_PTS // CHUNK
ROWS = 1000
LANES = 256

_CP = pltpu.CompilerParams(
    dimension_semantics=("parallel",),
    vmem_limit_bytes=100 * 1024 * 1024,
)


def _scatter_kernel(rh_ref, ri_ref, xi_ref, vh_ref, vi_ref, out_ref, acc):
    step = pl.program_id(0)

    @pl.when(step == 0)
    def _init():
        acc[...] = jnp.zeros_like(acc)

    iota = jax.lax.broadcasted_iota(jnp.int32, (1, LANES), 1)

    def body(i, _):
        for k in range(S):
            idx = i * S + k
            r = rh_ref[0, 0, idx]
            ri = ri_ref[0, 0, idx]
            c = xi_ref[0, 0, idx]
            vh = vh_ref[0, 0, idx]
            vi = vi_ref[0, 0, idx]
            onehot = iota == c
            row = acc[k, pl.ds(r, 1), :]
            acc[k, pl.ds(r, 1), :] = jnp.where(onehot, jnp.maximum(row, vh), row)
            row2 = acc[k, pl.ds(ri, 1), :]
            acc[k, pl.ds(ri, 1), :] = jnp.where(onehot, jnp.maximum(row2, vi), row2)
        return ()

    jax.lax.fori_loop(0, CHUNK // S, body, ())

    @pl.when(step == N_CHUNKS - 1)
    def _reduce():
        m01 = jnp.maximum(acc[0], acc[1])
        m23 = jnp.maximum(acc[2], acc[3])
        m45 = jnp.maximum(acc[4], acc[5])
        m67 = jnp.maximum(acc[6], acc[7])
        out_ref[...] = jnp.maximum(jnp.maximum(m01, m23), jnp.maximum(m45, m67))


def _points_to_bev_pallas(points):
    x, y, z, inten = points[:, 0], points[:, 1], points[:, 2], points[:, 3]
    xi = jnp.clip(jnp.floor((x - X0) / RES).astype(jnp.int32), 0, BEV_W - 1)
    yi = jnp.clip(jnp.floor((y - Y0) / RES).astype(jnp.int32), 0, BEV_H - 1)
    hi = ((z > -2.0).astype(jnp.int32) + (z > 0.0).astype(jnp.int32)
          + (z > 2.0).astype(jnp.int32) + (z > 4.0).astype(jnp.int32))
    hi = jnp.clip(hi, 0, 3)
    rh = (yi * 5 + hi).reshape(N_CHUNKS, 1, CHUNK)
    ri = (yi * 5 + 4).reshape(N_CHUNKS, 1, CHUNK)
    xi3 = xi.reshape(N_CHUNKS, 1, CHUNK)
    vh = (z + 2.0).reshape(N_CHUNKS, 1, CHUNK)
    vi = inten.reshape(N_CHUNKS, 1, CHUNK)

    smem = pl.BlockSpec((1, 1, CHUNK), lambda i: (i, 0, 0),
                        memory_space=pltpu.SMEM)
    grid_out = pl.pallas_call(
        _scatter_kernel,
        grid=(N_CHUNKS,),
        in_specs=[smem, smem, smem, smem, smem],
        out_specs=pl.BlockSpec((ROWS, LANES), lambda i: (0, 0)),
        out_shape=jax.ShapeDtypeStruct((ROWS, LANES), jnp.float32),
        scratch_shapes=[pltpu.VMEM((S, ROWS, LANES), jnp.float32)],
    )(rh, ri, xi3, vh, vi)
    return grid_out[:, :BEV_W].reshape(BEV_H, 5, BEV_W)


def _conv_s1_kernel(xp_ref, w_ref, b_ref, o_ref, *, th, cin, wo, ks, relu):
    i = pl.program_id(0)

    def row(r, _):
        h = i * th + r
        pieces = []
        for dy in range(ks):
            rowv = xp_ref[pl.ds(h + dy, 1)].reshape(cin, wo + ks - 1)
            for dx in range(ks):
                pieces.append(rowv[:, dx:dx + wo])
        xcat = pieces[0] if len(pieces) == 1 else jnp.concatenate(pieces, 0)
        acc = jnp.dot(w_ref[...], xcat, preferred_element_type=jnp.float32)
        acc = acc + b_ref[...][:, :1]
        if relu:
            acc = jnp.maximum(acc, 0.0)
        o_ref[pl.ds(r, 1)] = acc[None].astype(o_ref.dtype)
        return ()

    jax.lax.fori_loop(0, th, row, ())


def _conv_s1(xp, w, b, *, grid, relu, ks=3):
    hp, cin, wp = xp.shape
    ho, wo = hp - ks + 1, wp - ks + 1
    cout = w.shape[0]
    th = ho // grid
    bb = jnp.broadcast_to(b.astype(jnp.float32)[:, None], (cout, 128))
    return pl.pallas_call(
        functools.partial(_conv_s1_kernel, th=th, cin=cin, wo=wo, ks=ks,
                          relu=relu),
        grid=(grid,),
        in_specs=[pl.BlockSpec((hp, cin, wp), lambda i: (0, 0, 0)),
                  pl.BlockSpec(w.shape, lambda i: (0, 0)),
                  pl.BlockSpec((cout, 128), lambda i: (0, 0))],
        out_specs=pl.BlockSpec((th, cout, wo), lambda i: (i, 0, 0)),
        out_shape=jax.ShapeDtypeStruct((ho, cout, wo), jnp.bfloat16),
        compiler_params=_CP,
    )(xp.astype(jnp.bfloat16), w.astype(jnp.bfloat16), bb)


def _conv_s2_kernel(p00_ref, p01_ref, p10_ref, p11_ref, w_ref, b_ref, o_ref,
                    *, th, cin, w2, relu):
    i = pl.program_id(0)
    phases = {(0, 0): p00_ref, (0, 1): p01_ref,
              (1, 0): p10_ref, (1, 1): p11_ref}

    def row(r, _):
        h = i * th + r
        pieces = []
        for dy in range(3):
            py, oy = [(0, 0), (1, 0), (0, 1)][dy]
            rows = {px: phases[(py, px)][pl.ds(h + oy, 1)].reshape(cin, w2 + 1)
                    for px in (0, 1)}
            for dx in range(3):
                px, ox = [(0, 0), (1, 0), (0, 1)][dx]
                pieces.append(rows[px][:, ox:ox + w2])
        xcat = jnp.concatenate(pieces, 0)
        acc = jnp.dot(w_ref[...], xcat, preferred_element_type=jnp.float32)
        acc = acc + b_ref[...][:, :1]
        if relu:
            acc = jnp.maximum(acc, 0.0)
        o_ref[pl.ds(r, 1)] = acc[None].astype(o_ref.dtype)
        return ()

    jax.lax.fori_loop(0, th, row, ())


def _conv_s2(x, w, b, *, grid, relu):
    H, cin, W = x.shape
    h2, w2 = H // 2, W // 2
    cout = w.shape[0]
    th = h2 // grid
    xp = jnp.pad(x.astype(jnp.bfloat16), ((1, 1), (0, 0), (1, 1)))
    x4 = xp.reshape(H + 2, cin, (W + 2) // 2, 2)
    e, o = x4[:, :, :, 0], x4[:, :, :, 1]
    p00, p01 = e[0::2], o[0::2]
    p10, p11 = e[1::2], o[1::2]
    bb = jnp.broadcast_to(b.astype(jnp.float32)[:, None], (cout, 128))
    pspec = pl.BlockSpec((h2 + 1, cin, w2 + 1), lambda i: (0, 0, 0))
    return pl.pallas_call(
        functools.partial(_conv_s2_kernel, th=th, cin=cin, w2=w2, relu=relu),
        grid=(grid,),
        in_specs=[pspec, pspec, pspec, pspec,
                  pl.BlockSpec(w.shape, lambda i: (0, 0)),
                  pl.BlockSpec((cout, 128), lambda i: (0, 0))],
        out_specs=pl.BlockSpec((th, cout, w2), lambda i: (i, 0, 0)),
        out_shape=jax.ShapeDtypeStruct((h2, cout, w2), jnp.bfloat16),
        compiler_params=_CP,
    )(p00, p01, p10, p11, w.astype(jnp.bfloat16), bb)


def _rsw_kernel(x_ref, awt_ref, o_ref, *, th):
    i = pl.program_id(0)

    def row(r, _):
        xh = x_ref[pl.ds(i * th + r, 1)].reshape(128, 256)
        t = jnp.dot(xh, awt_ref[...], preferred_element_type=jnp.float32)
        o_ref[pl.ds(r, 1)] = t[None].astype(o_ref.dtype)
        return ()

    jax.lax.fori_loop(0, th, row, ())


def _resize_w(x, awt):
    return pl.pallas_call(
        functools.partial(_rsw_kernel, th=25),
        grid=(8,),
        in_specs=[pl.BlockSpec((200, 128, 256), lambda i: (0, 0, 0)),
                  pl.BlockSpec((256, 200), lambda i: (0, 0))],
        out_specs=pl.BlockSpec((25, 128, 200), lambda i: (i, 0, 0)),
        out_shape=jax.ShapeDtypeStruct((200, 128, 200), jnp.bfloat16),
        compiler_params=_CP,
    )(x.astype(jnp.bfloat16), awt.astype(jnp.bfloat16))


def _mmn_kernel(x_ref, w_ref, o_ref):
    o_ref[...] = jnp.dot(x_ref[...], w_ref[...],
                         preferred_element_type=jnp.float32).astype(o_ref.dtype)


def _mm_ngrid(x, w, *, tn, out_dtype=jnp.bfloat16):
    M, K = x.shape
    N = w.shape[1]
    return pl.pallas_call(
        _mmn_kernel,
        grid=(N // tn,),
        in_specs=[pl.BlockSpec((M, K), lambda i: (0, 0)),
                  pl.BlockSpec((K, tn), lambda i: (0, i))],
        out_specs=pl.BlockSpec((M, tn), lambda i: (0, i)),
        out_shape=jax.ShapeDtypeStruct((M, N), out_dtype),
        compiler_params=_CP,
    )(x.astype(jnp.bfloat16), w.astype(jnp.bfloat16))


def _fold_bn(conv_p, bn_p):
    s = bn_p["g"] * jax.lax.rsqrt(bn_p["v"] + EPS)
    w = conv_p["w"] * s[:, None, None, None]
    b = conv_p["b"] * s + bn_p["beta"] - bn_p["m"] * s
    return w, b


def _w_rows(w):
    return jnp.transpose(w, (0, 2, 3, 1)).reshape(w.shape[0], -1)


def _resize_mat(n_out, n_in):
    o = jnp.arange(n_out, dtype=jnp.float32)
    src = (o + 0.5) * (n_in / n_out) - 0.5
    i0 = jnp.floor(src)
    f = src - i0
    i0c = jnp.clip(i0.astype(jnp.int32), 0, n_in - 1)
    i1c = jnp.clip(i0.astype(jnp.int32) + 1, 0, n_in - 1)
    i = jnp.arange(n_in, dtype=jnp.int32)
    a = ((1.0 - f)[:, None] * (i[None, :] == i0c[:, None])
         + f[:, None] * (i[None, :] == i1c[:, None]))
    return a.astype(jnp.float32)


def _cam_branch(images, p):
    img = images[0].transpose(1, 0, 2).astype(jnp.bfloat16)

    w1, b1 = _fold_bn(p["c1"], p["bn1"])
    y = _conv_s2(img, _w_rows(w1), b1, grid=8, relu=True)

    w2, b2 = _fold_bn(p["c2"], p["bn2"])
    y = _conv_s2(y, _w_rows(w2), b2, grid=8, relu=True)

    w3, b3 = _fold_bn(p["c3"], p["bn3"])
    y = _conv_s2(y, _w_rows(w3), b3, grid=8, relu=True)

    wp1, bp1 = _fold_bn(p["p1"], p["pbn"])
    yp = jnp.pad(y, ((1, 1), (0, 0), (1, 1)))
    y = _conv_s1(yp, _w_rows(wp1), bp1, grid=8, relu=True)

    y = _conv_s1(y, p["p2"]["w"][:, :, 0, 0], p["p2"]["b"], grid=8,
                 relu=False, ks=1)

    ah = _resize_mat(BEV_H, 128)
    awt = _resize_mat(BEV_W, 256).T
    t = _mm_ngrid(ah, y.reshape(128, 128 * 256), tn=4096)
    return _resize_w(t.reshape(200, 128, 256), awt)


def _lid_branch(points, p):
    bev = _points_to_bev_pallas(points)

    w1, b1 = _fold_bn(p["c1"], p["bn1"])
    bp = jnp.pad(bev.astype(jnp.bfloat16), ((1, 1), (0, 0), (1, 1)))
    y = _conv_s1(bp, _w_rows(w1), b1, grid=8, relu=True)

    w2, b2 = _fold_bn(p["c2"], p["bn2"])
    yp = jnp.pad(y, ((1, 1), (0, 0), (1, 1)))
    y = _conv_s1(yp, _w_rows(w2), b2, grid=8, relu=True)

    y = _conv_s1(y, p["c3"]["w"][:, :, 0, 0], p["c3"]["b"], grid=8,
                 relu=False, ks=1)
    return y


def kernel(images, points, cam_params, lidar_params):
    cam = _cam_branch(images, cam_params)
    lid = _lid_branch(points, lidar_params)
    out = jnp.concatenate([cam, lid], axis=1)
    return out.transpose(1, 0, 2)[None].astype(jnp.float32)


# speedup vs baseline: 4.8672x; 1.0118x over previous
import functools

import jax
import jax.numpy as jnp
from jax.experimental import pallas as pl
from jax.experimental.pallas import tpu as pltpu

BEV_H, BEV_W = 200, 200
RES = 0.5
X0, Y0 = -50.0, -50.0
EPS = 1e-5

N_PTS = 20000
CHUNK = 2000
S = 16
N_CHUNKS = N_PTS // CHUNK
N_STEPS = N_CHUNKS // 2
ROWS = 1000
LANES = 256

_CP = pltpu.CompilerParams(
    dimension_semantics=("parallel",),
    vmem_limit_bytes=100 * 1024 * 1024,
)


def _scatter_kernel(rh_ref, ri_ref, xi_ref, vh_ref, vi_ref, out_ref, acc):
    step = pl.program_id(1)

    @pl.when(step == 0)
    def _init():
        acc[...] = jnp.zeros_like(acc)

    iota = jax.lax.broadcasted_iota(jnp.int32, (1, LANES), 1)

    def body(i, _):
        for k in range(S):
            idx = i * S + k
            r = rh_ref[0, 0, idx]
            ri = ri_ref[0, 0, idx]
            c = xi_ref[0, 0, idx]
            vh = vh_ref[0, 0, idx]
            vi = vi_ref[0, 0, idx]
            onehot = iota == c
            row = acc[k, pl.ds(r, 1), :]
            acc[k, pl.ds(r, 1), :] = jnp.where(onehot, jnp.maximum(row, vh), row)
            row2 = acc[k, pl.ds(ri, 1), :]
            acc[k, pl.ds(ri, 1), :] = jnp.where(onehot, jnp.maximum(row2, vi), row2)
        return ()

    jax.lax.fori_loop(0, CHUNK // S, body, ())

    @pl.when(step == N_STEPS - 1)
    def _reduce():
        m = acc[0]
        for k in range(1, S):
            m = jnp.maximum(m, acc[k])
        out_ref[...] = m[None]


def _points_to_bev_pallas(points):
    x, y, z, inten = points[:, 0], points[:, 1], points[:, 2], points[:, 3]
    xi = jnp.clip(jnp.floor((x - X0) / RES).astype(jnp.int32), 0, BEV_W - 1)
    yi = jnp.clip(jnp.floor((y - Y0) / RES).astype(jnp.int32), 0, BEV_H - 1)
    hi = ((z > -2.0).astype(jnp.int32) + (z > 0.0).astype(jnp.int32)
          + (z > 2.0).astype(jnp.int32) + (z > 4.0).astype(jnp.int32))
    hi = jnp.clip(hi, 0, 3)
    rh = (yi * 5 + hi).reshape(N_CHUNKS, 1, CHUNK)
    ri = (yi * 5 + 4).reshape(N_CHUNKS, 1, CHUNK)
    xi3 = xi.reshape(N_CHUNKS, 1, CHUNK)
    vh = (z + 2.0).reshape(N_CHUNKS, 1, CHUNK)
    vi = inten.reshape(N_CHUNKS, 1, CHUNK)

    smem = pl.BlockSpec((1, 1, CHUNK), lambda i, j: (i * N_STEPS + j, 0, 0),
                        memory_space=pltpu.SMEM)
    parts = pl.pallas_call(
        _scatter_kernel,
        grid=(2, N_STEPS),
        in_specs=[smem, smem, smem, smem, smem],
        out_specs=pl.BlockSpec((1, ROWS, LANES), lambda i, j: (i, 0, 0)),
        out_shape=jax.ShapeDtypeStruct((2, ROWS, LANES), jnp.float32),
        scratch_shapes=[pltpu.VMEM((S, ROWS, LANES), jnp.float32)],
        compiler_params=pltpu.CompilerParams(
            dimension_semantics=("parallel", "arbitrary"),
            vmem_limit_bytes=100 * 1024 * 1024),
    )(rh, ri, xi3, vh, vi)
    grid_out = jnp.maximum(parts[0], parts[1])
    return grid_out[:, :BEV_W].reshape(BEV_H, 5, BEV_W)


def _conv_s1_kernel(xp_ref, w_ref, b_ref, o_ref, *, th, cin, wo, ks, relu):
    i = pl.program_id(0)

    def row(r, _):
        h = i * th + r
        pieces = []
        for dy in range(ks):
            rowv = xp_ref[pl.ds(h + dy, 1)].reshape(cin, wo + ks - 1)
            for dx in range(ks):
                pieces.append(rowv[:, dx:dx + wo])
        xcat = pieces[0] if len(pieces) == 1 else jnp.concatenate(pieces, 0)
        acc = jnp.dot(w_ref[...], xcat, preferred_element_type=jnp.float32)
        acc = acc + b_ref[...][:, :1]
        if relu:
            acc = jnp.maximum(acc, 0.0)
        o_ref[pl.ds(r, 1)] = acc[None].astype(o_ref.dtype)
        return ()

    jax.lax.fori_loop(0, th, row, ())


def _conv_s1(xp, w, b, *, grid, relu, ks=3):
    hp, cin, wp = xp.shape
    ho, wo = hp - ks + 1, wp - ks + 1
    cout = w.shape[0]
    th = ho // grid
    bb = jnp.broadcast_to(b.astype(jnp.float32)[:, None], (cout, 128))
    return pl.pallas_call(
        functools.partial(_conv_s1_kernel, th=th, cin=cin, wo=wo, ks=ks,
                          relu=relu),
        grid=(grid,),
        in_specs=[pl.BlockSpec((hp, cin, wp), lambda i: (0, 0, 0)),
                  pl.BlockSpec(w.shape, lambda i: (0, 0)),
                  pl.BlockSpec((cout, 128), lambda i: (0, 0))],
        out_specs=pl.BlockSpec((th, cout, wo), lambda i: (i, 0, 0)),
        out_shape=jax.ShapeDtypeStruct((ho, cout, wo), jnp.bfloat16),
        compiler_params=_CP,
    )(xp.astype(jnp.bfloat16), w.astype(jnp.bfloat16), bb)


def _conv_s2_kernel(p00_ref, p01_ref, p10_ref, p11_ref, w_ref, b_ref, o_ref,
                    *, th, cin, w2, relu):
    i = pl.program_id(0)
    phases = {(0, 0): p00_ref, (0, 1): p01_ref,
              (1, 0): p10_ref, (1, 1): p11_ref}

    def row(r, _):
        h = i * th + r
        pieces = []
        for dy in range(3):
            py, oy = [(0, 0), (1, 0), (0, 1)][dy]
            rows = {px: phases[(py, px)][pl.ds(h + oy, 1)].reshape(cin, w2 + 1)
                    for px in (0, 1)}
            for dx in range(3):
                px, ox = [(0, 0), (1, 0), (0, 1)][dx]
                pieces.append(rows[px][:, ox:ox + w2])
        xcat = jnp.concatenate(pieces, 0)
        acc = jnp.dot(w_ref[...], xcat, preferred_element_type=jnp.float32)
        acc = acc + b_ref[...][:, :1]
        if relu:
            acc = jnp.maximum(acc, 0.0)
        o_ref[pl.ds(r, 1)] = acc[None].astype(o_ref.dtype)
        return ()

    jax.lax.fori_loop(0, th, row, ())


def _conv_s2(x, w, b, *, grid, relu):
    H, cin, W = x.shape
    h2, w2 = H // 2, W // 2
    cout = w.shape[0]
    th = h2 // grid
    xp = jnp.pad(x.astype(jnp.bfloat16), ((1, 1), (0, 0), (1, 1)))
    x4 = xp.reshape(H + 2, cin, (W + 2) // 2, 2)
    e, o = x4[:, :, :, 0], x4[:, :, :, 1]
    p00, p01 = e[0::2], o[0::2]
    p10, p11 = e[1::2], o[1::2]
    bb = jnp.broadcast_to(b.astype(jnp.float32)[:, None], (cout, 128))
    pspec = pl.BlockSpec((h2 + 1, cin, w2 + 1), lambda i: (0, 0, 0))
    return pl.pallas_call(
        functools.partial(_conv_s2_kernel, th=th, cin=cin, w2=w2, relu=relu),
        grid=(grid,),
        in_specs=[pspec, pspec, pspec, pspec,
                  pl.BlockSpec(w.shape, lambda i: (0, 0)),
                  pl.BlockSpec((cout, 128), lambda i: (0, 0))],
        out_specs=pl.BlockSpec((th, cout, w2), lambda i: (i, 0, 0)),
        out_shape=jax.ShapeDtypeStruct((h2, cout, w2), jnp.bfloat16),
        compiler_params=_CP,
    )(p00, p01, p10, p11, w.astype(jnp.bfloat16), bb)


def _rsw_kernel(x_ref, awt_ref, o_ref, *, th):
    i = pl.program_id(0)

    def row(r, _):
        xh = x_ref[pl.ds(i * th + r, 1)].reshape(128, 256)
        t = jnp.dot(xh, awt_ref[...], preferred_element_type=jnp.float32)
        o_ref[pl.ds(r, 1)] = t[None].astype(o_ref.dtype)
        return ()

    jax.lax.fori_loop(0, th, row, ())


def _resize_w(x, awt):
    return pl.pallas_call(
        functools.partial(_rsw_kernel, th=25),
        grid=(8,),
        in_specs=[pl.BlockSpec((200, 128, 256), lambda i: (0, 0, 0)),
                  pl.BlockSpec((256, 200), lambda i: (0, 0))],
        out_specs=pl.BlockSpec((25, 128, 200), lambda i: (i, 0, 0)),
        out_shape=jax.ShapeDtypeStruct((200, 128, 200), jnp.bfloat16),
        compiler_params=_CP,
    )(x.astype(jnp.bfloat16), awt.astype(jnp.bfloat16))


def _mmn_kernel(x_ref, w_ref, o_ref):
    o_ref[...] = jnp.dot(x_ref[...], w_ref[...],
                         preferred_element_type=jnp.float32).astype(o_ref.dtype)


def _mm_ngrid(x, w, *, tn, out_dtype=jnp.bfloat16):
    M, K = x.shape
    N = w.shape[1]
    return pl.pallas_call(
        _mmn_kernel,
        grid=(N // tn,),
        in_specs=[pl.BlockSpec((M, K), lambda i: (0, 0)),
                  pl.BlockSpec((K, tn), lambda i: (0, i))],
        out_specs=pl.BlockSpec((M, tn), lambda i: (0, i)),
        out_shape=jax.ShapeDtypeStruct((M, N), out_dtype),
        compiler_params=_CP,
    )(x.astype(jnp.bfloat16), w.astype(jnp.bfloat16))


def _fold_bn(conv_p, bn_p):
    s = bn_p["g"] * jax.lax.rsqrt(bn_p["v"] + EPS)
    w = conv_p["w"] * s[:, None, None, None]
    b = conv_p["b"] * s + bn_p["beta"] - bn_p["m"] * s
    return w, b


def _w_rows(w):
    return jnp.transpose(w, (0, 2, 3, 1)).reshape(w.shape[0], -1)


def _resize_mat(n_out, n_in):
    o = jnp.arange(n_out, dtype=jnp.float32)
    src = (o + 0.5) * (n_in / n_out) - 0.5
    i0 = jnp.floor(src)
    f = src - i0
    i0c = jnp.clip(i0.astype(jnp.int32), 0, n_in - 1)
    i1c = jnp.clip(i0.astype(jnp.int32) + 1, 0, n_in - 1)
    i = jnp.arange(n_in, dtype=jnp.int32)
    a = ((1.0 - f)[:, None] * (i[None, :] == i0c[:, None])
         + f[:, None] * (i[None, :] == i1c[:, None]))
    return a.astype(jnp.float32)


def _cam_branch(images, p):
    img = images[0].transpose(1, 0, 2).astype(jnp.bfloat16)

    w1, b1 = _fold_bn(p["c1"], p["bn1"])
    y = _conv_s2(img, _w_rows(w1), b1, grid=8, relu=True)

    w2, b2 = _fold_bn(p["c2"], p["bn2"])
    y = _conv_s2(y, _w_rows(w2), b2, grid=8, relu=True)

    w3, b3 = _fold_bn(p["c3"], p["bn3"])
    y = _conv_s2(y, _w_rows(w3), b3, grid=8, relu=True)

    wp1, bp1 = _fold_bn(p["p1"], p["pbn"])
    yp = jnp.pad(y, ((1, 1), (0, 0), (1, 1)))
    y = _conv_s1(yp, _w_rows(wp1), bp1, grid=8, relu=True)

    y = _conv_s1(y, p["p2"]["w"][:, :, 0, 0], p["p2"]["b"], grid=8,
                 relu=False, ks=1)

    ah = _resize_mat(BEV_H, 128)
    awt = _resize_mat(BEV_W, 256).T
    t = _mm_ngrid(ah, y.reshape(128, 128 * 256), tn=4096)
    return _resize_w(t.reshape(200, 128, 256), awt)


def _lid_branch(points, p):
    bev = _points_to_bev_pallas(points)

    w1, b1 = _fold_bn(p["c1"], p["bn1"])
    bp = jnp.pad(bev.astype(jnp.bfloat16), ((1, 1), (0, 0), (1, 1)))
    y = _conv_s1(bp, _w_rows(w1), b1, grid=8, relu=True)

    w2, b2 = _fold_bn(p["c2"], p["bn2"])
    yp = jnp.pad(y, ((1, 1), (0, 0), (1, 1)))
    y = _conv_s1(yp, _w_rows(w2), b2, grid=8, relu=True)

    y = _conv_s1(y, p["c3"]["w"][:, :, 0, 0], p["c3"]["b"], grid=8,
                 relu=False, ks=1)
    return y


def kernel(images, points, cam_params, lidar_params):
    cam = _cam_branch(images, cam_params)
    lid = _lid_branch(points, lidar_params)
    out = jnp.concatenate([cam, lid], axis=1)
    return out.transpose(1, 0, 2)[None].astype(jnp.float32)
